# baseline (speedup 1.0000x reference)
import functools

import jax
import jax.numpy as jnp
from jax import lax
from jax.experimental import pallas as pl
from jax.experimental.pallas import tpu as pltpu
from jax.experimental.pallas import tpu_sc as plsc

WPR = 72


def _coef_body(hist_ref, exits_ref, out_ref):
    h = hist_ref[...]
    ex = exits_ref[...]
    r, nh = h.shape
    cur = h[:, nh - 1 :]
    ise = jnp.any(h[:, :, None] == ex[:, None, :], axis=2)
    eq = h[:, :, None] == h[:, None, :]
    k_lt_j = (
        lax.broadcasted_iota(jnp.int32, (1, nh, nh), 2)
        < lax.broadcasted_iota(jnp.int32, (1, nh, nh), 1)
    )
    first = ~jnp.any(eq & k_lt_j, axis=2)
    col = lax.broadcasted_iota(jnp.int32, (r, nh), 1)
    is_cur_slot = col == nh - 1
    keep = (first & (h != cur)) | is_cur_slot
    vals = jnp.where(is_cur_slot, 0.5, 0.1) - ise.astype(jnp.float32)
    c_hist = jnp.where(keep, vals, 0.0)
    out_ref[...] = jnp.concatenate(
        [
            c_hist,
            jnp.ones((r, 16), jnp.float32),
            jnp.zeros((r, WPR - nh - 16), jnp.float32),
        ],
        axis=1,
    )


def _coefficients(hist, exits2d):
    B, H = hist.shape
    R = 128
    return pl.pallas_call(
        _coef_body,
        grid=(B // R,),
        in_specs=[
            pl.BlockSpec((R, H), lambda i: (i, 0)),
            pl.BlockSpec((1, 16), lambda i: (0, 0)),
        ],
        out_specs=pl.BlockSpec((R, WPR), lambda i: (i, 0)),
        out_shape=jax.ShapeDtypeStruct((B, WPR), jnp.float32),
    )(hist, exits2d)


def _bag(W1, idx_flat, coef_flat, B):
    D = W1.shape[1]
    info = plsc.get_sparse_core_info()
    NC, NS = info.num_cores, info.num_subcores
    NW = NC * NS
    rows_per_w = B // NW
    mesh = plsc.VectorSubcoreMesh(core_axis_name="c", subcore_axis_name="s")

    @functools.partial(
        pl.kernel,
        out_type=jax.ShapeDtypeStruct((B, D), jnp.float32),
        mesh=mesh,
        scratch_types=[
            pltpu.VMEM((rows_per_w * WPR,), jnp.int32),
            pltpu.VMEM((rows_per_w * WPR * 16,), jnp.float32),
            pltpu.VMEM((WPR, D), jnp.float32),
            pltpu.VMEM((WPR, D), jnp.float32),
            pltpu.VMEM((rows_per_w, D), jnp.float32),
            pltpu.SemaphoreType.DMA,
            pltpu.SemaphoreType.DMA,
        ],
    )
    def k(w1_hbm, idx_hbm, coef_hbm, out_hbm, idx_v, coef_v, buf_a, buf_b, out_v, sem_a, sem_b):
        wid = lax.axis_index("s") * NC + lax.axis_index("c")
        base_e = wid * rows_per_w * WPR

        pltpu.sync_copy(idx_hbm.at[pl.ds(base_e, rows_per_w * WPR)], idx_v)
        pltpu.sync_copy(coef_hbm.at[pl.ds(base_e * 16, rows_per_w * WPR * 16)], coef_v)

        bufs = (buf_a, buf_b)
        sems = (sem_a, sem_b)

        def fire(r, slot):
            pltpu.async_copy(
                w1_hbm.at[idx_v.at[pl.ds(r * WPR, WPR)]], bufs[slot], sems[slot]
            )

        def drain(r, slot):
            pltpu.make_async_copy(
                w1_hbm.at[idx_v.at[pl.ds(r * WPR, WPR)]], bufs[slot], sems[slot]
            ).wait()

        fire(0, 0)
        for r in range(rows_per_w):
            slot = r % 2
            if r + 1 < rows_per_w:
                fire(r + 1, 1 - slot)
            drain(r, slot)
            rows = bufs[slot]

            def body(j, accs):
                c = coef_v[pl.ds((r * WPR + j) * 16, 16)]
                return tuple(
                    accs[kk] + c * rows[j, pl.ds(kk * 16, 16)]
                    for kk in range(D // 16)
                )

            accs = lax.fori_loop(
                0,
                WPR - 6,
                body,
                tuple(jnp.zeros((16,), jnp.float32) for _ in range(D // 16)),
            )
            for kk in range(D // 16):
                out_v[r, pl.ds(kk * 16, 16)] = accs[kk]

        pltpu.sync_copy(out_v, out_hbm.at[pl.ds(wid * rows_per_w, rows_per_w)])

    return k(W1, idx_flat, coef_flat)


def _mlp_body(s_ref, b1_ref, w2_ref, b2_ref, out_ref):
    h = jnp.maximum(s_ref[...] + b1_ref[...], 0.0)
    o = lax.dot_general(
        h, w2_ref[...], (((1,), (0,)), ((), ())), preferred_element_type=jnp.float32
    )
    out_ref[...] = jnp.maximum(o + b2_ref[...], 0.0)


def _mlp(S, b1, W2, b2):
    B, D = S.shape
    O = W2.shape[1]
    return pl.pallas_call(
        _mlp_body,
        out_shape=jax.ShapeDtypeStruct((B, O), jnp.float32),
    )(S, b1.reshape(1, D), W2, b2.reshape(1, O))


def kernel(attacker_history, exits, W1, b1, W2, b2):
    hist = attacker_history.astype(jnp.int32)
    ex = exits.astype(jnp.int32)
    B, H = hist.shape
    idx = jnp.concatenate(
        [
            hist,
            jnp.broadcast_to(ex[None, :], (B, ex.shape[0])),
            jnp.zeros((B, WPR - H - ex.shape[0]), jnp.int32),
        ],
        axis=1,
    )
    coef = _coefficients(hist, ex.reshape(1, -1))
    coef_x = jnp.broadcast_to(coef[:, :, None], (B, WPR, 16))
    S = _bag(W1, idx.reshape(-1), coef_x.reshape(-1), B)
    return _mlp(S, b1, W2, b2)


# speedup vs baseline: 2.4412x; 2.4412x over previous
import functools

import jax
import jax.numpy as jnp
from jax import lax
from jax.experimental import pallas as pl
from jax.experimental.pallas import tpu as pltpu
from jax.experimental.pallas import tpu_sc as plsc

H = 50
WPR = 56
NBUF = 4


def _coef_body(hist_ref, exits_ref, out_ref):
    h = hist_ref[...]
    ex = exits_ref[...]
    r, nh = h.shape
    cur = h[:, nh - 1 :]
    ise = jnp.any(h[:, :, None] == ex[:, None, :], axis=2)
    eq = h[:, :, None] == h[:, None, :]
    k_lt_j = (
        lax.broadcasted_iota(jnp.int32, (1, nh, nh), 2)
        < lax.broadcasted_iota(jnp.int32, (1, nh, nh), 1)
    )
    first = ~jnp.any(eq & k_lt_j, axis=2)
    col = lax.broadcasted_iota(jnp.int32, (r, nh), 1)
    is_cur_slot = col == nh - 1
    keep = (first & (h != cur)) | is_cur_slot
    vals = jnp.where(is_cur_slot, 0.5, 0.1) - ise.astype(jnp.float32)
    c_hist = jnp.where(keep, vals, 0.0)
    out_ref[...] = jnp.concatenate(
        [c_hist, jnp.zeros((r, WPR - nh), jnp.float32)], axis=1
    )


def _coefficients(hist, exits2d):
    B, nh = hist.shape
    R = 128
    return pl.pallas_call(
        _coef_body,
        grid=(B // R,),
        in_specs=[
            pl.BlockSpec((R, nh), lambda i: (i, 0)),
            pl.BlockSpec((1, 16), lambda i: (0, 0)),
        ],
        out_specs=pl.BlockSpec((R, WPR), lambda i: (i, 0)),
        out_shape=jax.ShapeDtypeStruct((B, WPR), jnp.float32),
    )(hist, exits2d)


def _bag(W1, exits, idx_flat, coef_flat, B):
    D = W1.shape[1]
    NE = exits.shape[0]
    DC = D // 16
    info = plsc.get_sparse_core_info()
    NC, NS = info.num_cores, info.num_subcores
    NW = NC * NS
    RPW = B // NW
    GROUPS = RPW // NBUF
    mesh = plsc.VectorSubcoreMesh(core_axis_name="c", subcore_axis_name="s")

    @functools.partial(
        pl.kernel,
        out_type=jax.ShapeDtypeStruct((B, D), jnp.float32),
        mesh=mesh,
        scratch_types=[
            pltpu.VMEM((RPW * WPR,), jnp.int32),
            pltpu.VMEM((RPW * WPR * 16,), jnp.float32),
            pltpu.VMEM((NE,), jnp.int32),
            pltpu.VMEM((NE, D), jnp.float32),
            [pltpu.VMEM((H, D), jnp.float32) for _ in range(NBUF)],
            pltpu.VMEM((RPW, D), jnp.float32),
            [pltpu.SemaphoreType.DMA for _ in range(NBUF)],
            pltpu.SemaphoreType.DMA,
        ],
    )
    def k(w1_hbm, ex_hbm, idx_hbm, coef_hbm, out_hbm,
          idx_v, coef_v, ex_v, ebuf, bufs, out_v, sems, esem):
        wid = lax.axis_index("s") * NC + lax.axis_index("c")
        base_e = wid * RPW * WPR

        pltpu.sync_copy(idx_hbm.at[pl.ds(base_e, RPW * WPR)], idx_v)
        pltpu.sync_copy(coef_hbm.at[pl.ds(base_e * 16, RPW * WPR * 16)], coef_v)
        pltpu.sync_copy(ex_hbm, ex_v)
        pltpu.async_copy(w1_hbm.at[ex_v], ebuf, esem).wait()

        e_acc = []
        for kk in range(DC):
            s = ebuf[0, pl.ds(kk * 16, 16)]
            for e in range(1, NE):
                s = s + ebuf[e, pl.ds(kk * 16, 16)]
            e_acc.append(s)

        def fire(row, slot):
            pltpu.async_copy(
                w1_hbm.at[idx_v.at[pl.ds(row * WPR, H)]], bufs[slot], sems[slot]
            )

        def drain(row, slot):
            pltpu.make_async_copy(
                w1_hbm.at[idx_v.at[pl.ds(row * WPR, H)]], bufs[slot], sems[slot]
            ).wait()

        for b in range(NBUF):
            fire(b, b)

        def outer(g, e):
            for b in range(NBUF):
                row = g * NBUF + b
                drain(row, b)
                accs = list(e)
                for j in range(H):
                    c = coef_v[pl.ds((row * WPR + j) * 16, 16)]
                    for kk in range(DC):
                        accs[kk] = accs[kk] + c * bufs[b][j, pl.ds(kk * 16, 16)]
                for kk in range(DC):
                    out_v[row, pl.ds(kk * 16, 16)] = accs[kk]

                @pl.when(row + NBUF < RPW)
                def _():
                    fire(row + NBUF, b)

            return e

        lax.fori_loop(0, GROUPS, outer, tuple(e_acc))
        pltpu.sync_copy(out_v, out_hbm.at[pl.ds(wid * RPW, RPW)])

    return k(W1, exits, idx_flat, coef_flat)


def _mlp_body(s_ref, b1_ref, w2_ref, b2_ref, out_ref):
    h = jnp.maximum(s_ref[...] + b1_ref[...], 0.0)
    o = lax.dot_general(
        h, w2_ref[...], (((1,), (0,)), ((), ())), preferred_element_type=jnp.float32
    )
    out_ref[...] = jnp.maximum(o + b2_ref[...], 0.0)


def _mlp(S, b1, W2, b2):
    B, D = S.shape
    O = W2.shape[1]
    return pl.pallas_call(
        _mlp_body,
        out_shape=jax.ShapeDtypeStruct((B, O), jnp.float32),
    )(S, b1.reshape(1, D), W2, b2.reshape(1, O))


def kernel(attacker_history, exits, W1, b1, W2, b2):
    hist = attacker_history.astype(jnp.int32)
    ex = exits.astype(jnp.int32)
    B, nh = hist.shape
    idx = jnp.concatenate([hist, jnp.zeros((B, WPR - nh), jnp.int32)], axis=1)
    coef = _coefficients(hist, ex.reshape(1, -1))
    coef_x = jnp.broadcast_to(coef[:, :, None], (B, WPR, 16))
    S = _bag(W1, ex, idx.reshape(-1), coef_x.reshape(-1), B)
    return _mlp(S, b1, W2, b2)


# speedup vs baseline: 3.9905x; 1.6346x over previous
import functools

import jax
import jax.numpy as jnp
from jax import lax
from jax.experimental import pallas as pl
from jax.experimental.pallas import tpu as pltpu
from jax.experimental.pallas import tpu_sc as plsc

H = 50
WPR = 56
NBUF = 4


def _coef_body(hist_ref, exits_ref, out_ref):
    h = hist_ref[...]
    ex = exits_ref[...]
    r, nh = h.shape
    cur = h[:, nh - 1 :]
    ise = h == ex[:, 0:1]
    for e in range(1, 16):
        ise = ise | (h == ex[:, e : e + 1])
    hp = jnp.concatenate([h, jnp.full((r, 64 - nh), -1, jnp.int32)], axis=1)
    col64 = lax.broadcasted_iota(jnp.int32, (r, 64), 1)
    dup = jnp.zeros((r, 64), jnp.bool_)
    for d in range(1, nh):
        dup = dup | ((hp == jnp.roll(hp, d, axis=1)) & (col64 >= d))
    first = ~dup[:, :nh]
    col = lax.broadcasted_iota(jnp.int32, (r, nh), 1)
    is_cur_slot = col == nh - 1
    keep = (first & (h != cur)) | is_cur_slot
    vals = jnp.where(is_cur_slot, 0.5, 0.1) - ise.astype(jnp.float32)
    c_hist = jnp.where(keep, vals, 0.0)
    out_ref[...] = jnp.concatenate(
        [c_hist, jnp.zeros((r, WPR - nh), jnp.float32)], axis=1
    )


def _coefficients(hist, exits2d):
    B, nh = hist.shape
    R = 128
    return pl.pallas_call(
        _coef_body,
        grid=(B // R,),
        in_specs=[
            pl.BlockSpec((R, nh), lambda i: (i, 0)),
            pl.BlockSpec((1, 16), lambda i: (0, 0)),
        ],
        out_specs=pl.BlockSpec((R, WPR), lambda i: (i, 0)),
        out_shape=jax.ShapeDtypeStruct((B, WPR), jnp.float32),
    )(hist, exits2d)


def _bag(W1, exits, idx_flat, coef_flat, B):
    D = W1.shape[1]
    NE = exits.shape[0]
    DC = D // 16
    info = plsc.get_sparse_core_info()
    NC, NS = info.num_cores, info.num_subcores
    NW = NC * NS
    RPW = B // NW
    GROUPS = RPW // NBUF
    mesh = plsc.VectorSubcoreMesh(core_axis_name="c", subcore_axis_name="s")

    @functools.partial(
        pl.kernel,
        out_type=jax.ShapeDtypeStruct((B, D), jnp.float32),
        mesh=mesh,
        scratch_types=[
            pltpu.VMEM((RPW * WPR,), jnp.int32),
            pltpu.VMEM((RPW * WPR + 16,), jnp.float32),
            pltpu.VMEM((NE,), jnp.int32),
            pltpu.VMEM((NE, D), jnp.float32),
            [pltpu.VMEM((H, D), jnp.float32) for _ in range(NBUF)],
            pltpu.VMEM((RPW, D), jnp.float32),
            [pltpu.SemaphoreType.DMA for _ in range(NBUF)],
            pltpu.SemaphoreType.DMA,
        ],
    )
    def k(w1_hbm, ex_hbm, idx_hbm, coef_hbm, out_hbm,
          idx_v, coef_v, ex_v, ebuf, bufs, out_v, sems, esem):
        wid = lax.axis_index("s") * NC + lax.axis_index("c")
        base_e = wid * RPW * WPR

        pltpu.sync_copy(idx_hbm.at[pl.ds(base_e, RPW * WPR)], idx_v)
        pltpu.sync_copy(
            coef_hbm.at[pl.ds(base_e, RPW * WPR)],
            coef_v.at[pl.ds(0, RPW * WPR)],
        )
        pltpu.sync_copy(ex_hbm, ex_v)
        pltpu.async_copy(w1_hbm.at[ex_v], ebuf, esem).wait()

        e_acc = []
        for kk in range(DC):
            s = ebuf[0, pl.ds(kk * 16, 16)]
            for e in range(1, NE):
                s = s + ebuf[e, pl.ds(kk * 16, 16)]
            e_acc.append(s)

        def fire(row, slot):
            pltpu.async_copy(
                w1_hbm.at[idx_v.at[pl.ds(row * WPR, H)]], bufs[slot], sems[slot]
            )

        def drain(row, slot):
            pltpu.make_async_copy(
                w1_hbm.at[idx_v.at[pl.ds(row * WPR, H)]], bufs[slot], sems[slot]
            ).wait()

        for b in range(NBUF):
            fire(b, b)

        def outer(g, e):
            for b in range(NBUF):
                row = g * NBUF + b
                drain(row, b)
                accs = list(e)
                for j in range(H):
                    c = coef_v[pl.ds(row * WPR + j, 16)][0]
                    for kk in range(DC):
                        accs[kk] = accs[kk] + c * bufs[b][j, pl.ds(kk * 16, 16)]
                for kk in range(DC):
                    out_v[row, pl.ds(kk * 16, 16)] = accs[kk]

                @pl.when(row + NBUF < RPW)
                def _():
                    fire(row + NBUF, b)

            return e

        lax.fori_loop(0, GROUPS, outer, tuple(e_acc))
        pltpu.sync_copy(out_v, out_hbm.at[pl.ds(wid * RPW, RPW)])

    return k(W1, exits, idx_flat, coef_flat)


def _mlp_body(s_ref, b1_ref, w2_ref, b2_ref, out_ref):
    h = jnp.maximum(s_ref[...] + b1_ref[...], 0.0)
    o = lax.dot_general(
        h, w2_ref[...], (((1,), (0,)), ((), ())), preferred_element_type=jnp.float32
    )
    out_ref[...] = jnp.maximum(o + b2_ref[...], 0.0)


def _mlp(S, b1, W2, b2):
    B, D = S.shape
    O = W2.shape[1]
    return pl.pallas_call(
        _mlp_body,
        out_shape=jax.ShapeDtypeStruct((B, O), jnp.float32),
    )(S, b1.reshape(1, D), W2, b2.reshape(1, O))


def kernel(attacker_history, exits, W1, b1, W2, b2):
    hist = attacker_history.astype(jnp.int32)
    ex = exits.astype(jnp.int32)
    B, nh = hist.shape
    idx = jnp.concatenate([hist, jnp.zeros((B, WPR - nh), jnp.int32)], axis=1)
    coef = _coefficients(hist, ex.reshape(1, -1))
    S = _bag(W1, ex, idx.reshape(-1), coef.reshape(-1), B)
    return _mlp(S, b1, W2, b2)


# speedup vs baseline: 5.0102x; 1.2555x over previous
import functools

import jax
import jax.numpy as jnp
from jax import lax
from jax.experimental import pallas as pl
from jax.experimental.pallas import tpu as pltpu
from jax.experimental.pallas import tpu_sc as plsc

H = 50
WPR = 56
NBUF = 4


def _coef_body(hist_ref, exits_ref, out_ref):
    ht = hist_ref[...]
    nh = ht.shape[0]
    cur = ht[nh - 1]
    ise = ht == exits_ref[0]
    for e in range(1, 16):
        ise = ise | (ht == exits_ref[e])
    slots = []
    for j in range(nh):
        hj = ht[j]
        if j == nh - 1:
            c = 0.5 - ise[j].astype(jnp.float32)
        else:
            dup = jnp.zeros(hj.shape, jnp.bool_) if j == 0 else jnp.any(
                ht[:j] == hj[None], axis=0
            )
            keep = ~dup & (hj != cur)
            c = jnp.where(keep, 0.1 - ise[j].astype(jnp.float32), 0.0)
        slots.append(c)
    for _ in range(WPR - nh):
        slots.append(jnp.zeros(cur.shape, jnp.float32))
    out_ref[...] = jnp.stack(slots, axis=0)


def _coefficients(hist, exits):
    B, nh = hist.shape
    hist_t = hist.T.reshape(nh, B // 128, 128)
    coef_t = pl.pallas_call(
        _coef_body,
        in_specs=[
            pl.BlockSpec(memory_space=pltpu.VMEM),
            pl.BlockSpec(memory_space=pltpu.SMEM),
        ],
        out_shape=jax.ShapeDtypeStruct((WPR, B // 128, 128), jnp.float32),
    )(hist_t, exits)
    return coef_t.reshape(WPR, B).T


def _bag(W1, exits, idx_flat, coef_flat, B):
    D = W1.shape[1]
    NE = exits.shape[0]
    DC = D // 16
    info = plsc.get_sparse_core_info()
    NC, NS = info.num_cores, info.num_subcores
    NW = NC * NS
    RPW = B // NW
    GROUPS = RPW // NBUF
    mesh = plsc.VectorSubcoreMesh(core_axis_name="c", subcore_axis_name="s")

    @functools.partial(
        pl.kernel,
        out_type=jax.ShapeDtypeStruct((B, D), jnp.float32),
        mesh=mesh,
        scratch_types=[
            pltpu.VMEM((RPW * WPR,), jnp.int32),
            pltpu.VMEM((RPW * WPR + 16,), jnp.float32),
            pltpu.VMEM((NE,), jnp.int32),
            pltpu.VMEM((NE, D), jnp.float32),
            [pltpu.VMEM((H, D), jnp.float32) for _ in range(NBUF)],
            pltpu.VMEM((RPW, D), jnp.float32),
            [pltpu.SemaphoreType.DMA for _ in range(NBUF)],
            pltpu.SemaphoreType.DMA,
        ],
    )
    def k(w1_hbm, ex_hbm, idx_hbm, coef_hbm, out_hbm,
          idx_v, coef_v, ex_v, ebuf, bufs, out_v, sems, esem):
        wid = lax.axis_index("s") * NC + lax.axis_index("c")
        base_e = wid * RPW * WPR

        pltpu.sync_copy(idx_hbm.at[pl.ds(base_e, RPW * WPR)], idx_v)
        pltpu.sync_copy(
            coef_hbm.at[pl.ds(base_e, RPW * WPR)],
            coef_v.at[pl.ds(0, RPW * WPR)],
        )
        pltpu.sync_copy(ex_hbm, ex_v)
        pltpu.async_copy(w1_hbm.at[ex_v], ebuf, esem).wait()

        e_acc = []
        for kk in range(DC):
            s = ebuf[0, pl.ds(kk * 16, 16)]
            for e in range(1, NE):
                s = s + ebuf[e, pl.ds(kk * 16, 16)]
            e_acc.append(s)

        def fire(row, slot):
            pltpu.async_copy(
                w1_hbm.at[idx_v.at[pl.ds(row * WPR, H)]], bufs[slot], sems[slot]
            )

        def drain(row, slot):
            pltpu.make_async_copy(
                w1_hbm.at[idx_v.at[pl.ds(row * WPR, H)]], bufs[slot], sems[slot]
            ).wait()

        for b in range(NBUF):
            fire(b, b)

        def outer(g, e):
            for b in range(NBUF):
                row = g * NBUF + b
                drain(row, b)
                accs = list(e)
                for j in range(H):
                    c = coef_v[pl.ds(row * WPR + j, 16)][0]
                    for kk in range(DC):
                        accs[kk] = accs[kk] + c * bufs[b][j, pl.ds(kk * 16, 16)]
                for kk in range(DC):
                    out_v[row, pl.ds(kk * 16, 16)] = accs[kk]

                @pl.when(row + NBUF < RPW)
                def _():
                    fire(row + NBUF, b)

            return e

        lax.fori_loop(0, GROUPS, outer, tuple(e_acc))
        pltpu.sync_copy(out_v, out_hbm.at[pl.ds(wid * RPW, RPW)])

    return k(W1, exits, idx_flat, coef_flat)


def _mlp_body(s_ref, b1_ref, w2_ref, b2_ref, out_ref):
    h = jnp.maximum(s_ref[...] + b1_ref[...], 0.0)
    o = lax.dot_general(
        h, w2_ref[...], (((1,), (0,)), ((), ())), preferred_element_type=jnp.float32
    )
    out_ref[...] = jnp.maximum(o + b2_ref[...], 0.0)


def _mlp(S, b1, W2, b2):
    B, D = S.shape
    O = W2.shape[1]
    return pl.pallas_call(
        _mlp_body,
        out_shape=jax.ShapeDtypeStruct((B, O), jnp.float32),
    )(S, b1.reshape(1, D), W2, b2.reshape(1, O))


def kernel(attacker_history, exits, W1, b1, W2, b2):
    hist = attacker_history.astype(jnp.int32)
    ex = exits.astype(jnp.int32)
    B, nh = hist.shape
    idx = jnp.concatenate([hist, jnp.zeros((B, WPR - nh), jnp.int32)], axis=1)
    coef = _coefficients(hist, ex)
    S = _bag(W1, ex, idx.reshape(-1), coef.reshape(-1), B)
    return _mlp(S, b1, W2, b2)
